# baseline (speedup 1.0000x reference)
import functools

import jax
import jax.numpy as jnp
from jax import lax
from jax.experimental import pallas as pl
from jax.experimental.pallas import tpu as pltpu
from jax.experimental.pallas import tpu_sc as plsc

_DIM = 256
_K = 8192
_N = 16384
_BM = 512
_BN = 512
_NCHUNK = _K // _BN


def _vq_body(iota_ref, x_ref, emb_ref, idx_ref, msum_ref, emb2_ref, e2_ref):
    i = pl.program_id(0)

    @pl.when(i == 0)
    def _():
        emb2_ref[...] = emb_ref[...] + emb_ref[...]
        sq = emb_ref[...] * emb_ref[...]
        e2_ref[...] = lax.dot_general(
            jnp.ones((1, _DIM), jnp.float32), sq, (((1,), (1,)), ((), ())),
            preferred_element_type=jnp.float32)

    x_blk = x_ref[...]
    x2 = jnp.sum(x_blk * x_blk, axis=1, keepdims=True)
    iof = iota_ref[...]

    def scan(add_e2):
        accv = None
        acci = None
        for j in range(_NCHUNK):
            e_blk = emb2_ref[pl.ds(j * _BN, _BN), :]
            mm2 = lax.dot_general(x_blk, e_blk, (((1,), (1,)), ((), ())),
                                  preferred_element_type=jnp.float32)
            if add_e2:
                t = x2 + e2_ref[:, pl.ds(j * _BN, _BN)]
            else:
                t = x2
            s = t - mm2
            s_lo = s[:, : _BN // 2]
            s_hi = s[:, _BN // 2:]
            fold = s_hi < s_lo
            sv = jnp.where(fold, s_hi, s_lo)
            si = jnp.where(fold, iof + jnp.float32(j * _BN + _BN // 2),
                           iof + jnp.float32(j * _BN))
            if accv is None:
                accv = sv
                acci = si
            else:
                take = sv < accv
                accv = jnp.where(take, sv, accv)
                acci = jnp.where(take, si, acci)
        m = jnp.min(accv, axis=1, keepdims=True)
        cand = jnp.where(accv == m, acci, jnp.float32(1e9))
        ci = jnp.min(cand, axis=1, keepdims=True)
        idx_ref[...] = ci[:, 0].astype(jnp.int32)
        part = jnp.sum(m)

        @pl.when(i == 0)
        def _():
            msum_ref[0, 0] = part

        @pl.when(i != 0)
        def _():
            msum_ref[0, 0] = msum_ref[0, 0] + part

    safe = jnp.max(e2_ref[...]) * jnp.float32(1 << 25) < jnp.min(x2)

    @pl.when(safe)
    def _():
        scan(False)

    @pl.when(jnp.logical_not(safe))
    def _():
        scan(True)


def _vq_argmin(iota_f, x, embedding):
    return pl.pallas_call(
        _vq_body,
        grid=(_N // _BM,),
        in_specs=[
            pl.BlockSpec((1, _BN // 2), lambda i: (0, 0)),
            pl.BlockSpec((_BM, _DIM), lambda i: (i, 0)),
            pl.BlockSpec((_K, _DIM), lambda i: (0, 0)),
        ],
        out_specs=[
            pl.BlockSpec((_BM,), lambda i: (i,)),
            pl.BlockSpec((1, 1), lambda i: (0, 0), memory_space=pltpu.SMEM),
        ],
        out_shape=[
            jax.ShapeDtypeStruct((_N,), jnp.int32),
            jax.ShapeDtypeStruct((1, 1), jnp.float32),
        ],
        scratch_shapes=[
            pltpu.VMEM((_K, _DIM), jnp.float32),
            pltpu.VMEM((1, _K), jnp.float32),
        ],
    )(iota_f, x, embedding)


def _sc_gather(embedding, indices):
    info = plsc.get_sparse_core_info()
    nc, ns = info.num_cores, info.num_subcores
    nw = nc * ns
    bpw = _N // nw
    ch = 128
    nch = bpw // ch
    mesh = plsc.VectorSubcoreMesh(core_axis_name="c", subcore_axis_name="s")

    @functools.partial(
        pl.kernel,
        mesh=mesh,
        out_type=jax.ShapeDtypeStruct((_N, _DIM), jnp.float32),
        scratch_types=[
            pltpu.VMEM((bpw,), jnp.int32),
            pltpu.VMEM((ch, _DIM), jnp.float32),
            pltpu.VMEM((ch, _DIM), jnp.float32),
            pltpu.SemaphoreType.DMA,
            pltpu.SemaphoreType.DMA,
        ],
    )
    def gk(table_hbm, idx_hbm, out_hbm, idx_v, rows0, rows1, sem0, sem1):
        wid = lax.axis_index("s") * nc + lax.axis_index("c")
        base = wid * bpw
        pltpu.sync_copy(idx_hbm.at[pl.ds(base, bpw)], idx_v)
        bufs = (rows0, rows1)
        sems = (sem0, sem1)
        prev = pltpu.async_copy(table_hbm.at[idx_v.at[pl.ds(0, ch)]],
                                bufs[0], sems[0])
        for c in range(1, nch):
            nxt = pltpu.async_copy(table_hbm.at[idx_v.at[pl.ds(c * ch, ch)]],
                                   bufs[c % 2], sems[c % 2])
            prev.wait()
            pltpu.sync_copy(bufs[(c - 1) % 2],
                            out_hbm.at[pl.ds(base + (c - 1) * ch, ch)])
            prev = nxt
        prev.wait()
        pltpu.sync_copy(bufs[(nch - 1) % 2],
                        out_hbm.at[pl.ds(base + (nch - 1) * ch, ch)])

    return gk(embedding, indices)


def kernel(x, embedding):
    input_shape = x.shape
    flat_x = x.reshape(-1, _DIM)
    iota_f = jnp.arange(_BN // 2, dtype=jnp.float32)[None, :]
    indices, msum = _vq_argmin(iota_f, flat_x, embedding)
    q = _sc_gather(embedding, indices)
    commitment_loss = msum[0, 0] / (_N * _DIM)
    quantized = q.reshape(input_shape)
    return quantized, indices, commitment_loss


# speedup vs baseline: 1.1294x; 1.1294x over previous
import functools

import jax
import jax.numpy as jnp
from jax import lax
from jax.experimental import pallas as pl
from jax.experimental.pallas import tpu as pltpu
from jax.experimental.pallas import tpu_sc as plsc

_DIM = 256
_K = 8192
_N = 16384
_BM = 1024
_BN = 256
_NCHUNK = _K // _BN


def _vq_body(iota_ref, x_ref, emb_ref, idx_ref, msum_ref, emb2_ref, e2_ref):
    i = pl.program_id(0)

    @pl.when(i == 0)
    def _():
        emb2_ref[...] = emb_ref[...] + emb_ref[...]
        sq = emb_ref[...] * emb_ref[...]
        e2_ref[...] = lax.dot_general(
            jnp.ones((1, _DIM), jnp.float32), sq, (((1,), (1,)), ((), ())),
            preferred_element_type=jnp.float32)

    x_blk = x_ref[...]
    x2 = jnp.sum(x_blk * x_blk, axis=1, keepdims=True)
    iof = iota_ref[...]

    def scan(add_e2):
        accv = None
        acci = None
        for j in range(_NCHUNK):
            e_blk = emb2_ref[pl.ds(j * _BN, _BN), :]
            mm2 = lax.dot_general(x_blk, e_blk, (((1,), (1,)), ((), ())),
                                  preferred_element_type=jnp.float32)
            if add_e2:
                t = x2 + e2_ref[:, pl.ds(j * _BN, _BN)]
            else:
                t = x2
            s = t - mm2
            s_lo = s[:, : _BN // 2]
            s_hi = s[:, _BN // 2:]
            fold = s_hi < s_lo
            sv = jnp.where(fold, s_hi, s_lo)
            si = jnp.where(fold, iof + jnp.float32(j * _BN + _BN // 2),
                           iof + jnp.float32(j * _BN))
            if accv is None:
                accv = sv
                acci = si
            else:
                take = sv < accv
                accv = jnp.where(take, sv, accv)
                acci = jnp.where(take, si, acci)
        m = jnp.min(accv, axis=1, keepdims=True)
        cand = jnp.where(accv == m, acci, jnp.float32(1e9))
        ci = jnp.min(cand, axis=1, keepdims=True)
        idx_ref[...] = ci[:, 0].astype(jnp.int32)
        part = jnp.sum(m)

        @pl.when(i == 0)
        def _():
            msum_ref[0, 0] = part

        @pl.when(i != 0)
        def _():
            msum_ref[0, 0] = msum_ref[0, 0] + part

    safe = jnp.max(e2_ref[...]) * jnp.float32(1 << 25) < jnp.min(x2)

    @pl.when(safe)
    def _():
        scan(False)

    @pl.when(jnp.logical_not(safe))
    def _():
        scan(True)


def _vq_argmin(iota_f, x, embedding):
    return pl.pallas_call(
        _vq_body,
        grid=(_N // _BM,),
        in_specs=[
            pl.BlockSpec((1, _BN // 2), lambda i: (0, 0)),
            pl.BlockSpec((_BM, _DIM), lambda i: (i, 0)),
            pl.BlockSpec((_K, _DIM), lambda i: (0, 0)),
        ],
        out_specs=[
            pl.BlockSpec((_BM,), lambda i: (i,)),
            pl.BlockSpec((1, 1), lambda i: (0, 0), memory_space=pltpu.SMEM),
        ],
        out_shape=[
            jax.ShapeDtypeStruct((_N,), jnp.int32),
            jax.ShapeDtypeStruct((1, 1), jnp.float32),
        ],
        scratch_shapes=[
            pltpu.VMEM((_K, _DIM), jnp.float32),
            pltpu.VMEM((1, _K), jnp.float32),
        ],
    )(iota_f, x, embedding)


def _sc_gather(embedding, indices):
    info = plsc.get_sparse_core_info()
    nc, ns = info.num_cores, info.num_subcores
    nw = nc * ns
    bpw = _N // nw
    ch = 128
    nch = bpw // ch
    mesh = plsc.VectorSubcoreMesh(core_axis_name="c", subcore_axis_name="s")

    @functools.partial(
        pl.kernel,
        mesh=mesh,
        out_type=jax.ShapeDtypeStruct((_N, _DIM), jnp.float32),
        scratch_types=[
            pltpu.VMEM((bpw,), jnp.int32),
            pltpu.VMEM((ch, _DIM), jnp.float32),
            pltpu.VMEM((ch, _DIM), jnp.float32),
            pltpu.SemaphoreType.DMA,
            pltpu.SemaphoreType.DMA,
        ],
    )
    def gk(table_hbm, idx_hbm, out_hbm, idx_v, rows0, rows1, sem0, sem1):
        wid = lax.axis_index("s") * nc + lax.axis_index("c")
        base = wid * bpw
        pltpu.sync_copy(idx_hbm.at[pl.ds(base, bpw)], idx_v)
        bufs = (rows0, rows1)
        sems = (sem0, sem1)
        prev = pltpu.async_copy(table_hbm.at[idx_v.at[pl.ds(0, ch)]],
                                bufs[0], sems[0])
        for c in range(1, nch):
            nxt = pltpu.async_copy(table_hbm.at[idx_v.at[pl.ds(c * ch, ch)]],
                                   bufs[c % 2], sems[c % 2])
            prev.wait()
            pltpu.sync_copy(bufs[(c - 1) % 2],
                            out_hbm.at[pl.ds(base + (c - 1) * ch, ch)])
            prev = nxt
        prev.wait()
        pltpu.sync_copy(bufs[(nch - 1) % 2],
                        out_hbm.at[pl.ds(base + (nch - 1) * ch, ch)])

    return gk(embedding, indices)


def kernel(x, embedding):
    input_shape = x.shape
    flat_x = x.reshape(-1, _DIM)
    iota_f = jnp.arange(_BN // 2, dtype=jnp.float32)[None, :]
    indices, msum = _vq_argmin(iota_f, flat_x, embedding)
    q = _sc_gather(embedding, indices)
    commitment_loss = msum[0, 0] / (_N * _DIM)
    quantized = q.reshape(input_shape)
    return quantized, indices, commitment_loss


# speedup vs baseline: 1.1542x; 1.0219x over previous
import functools

import jax
import jax.numpy as jnp
from jax import lax
from jax.experimental import pallas as pl
from jax.experimental.pallas import tpu as pltpu
from jax.experimental.pallas import tpu_sc as plsc

_DIM = 256
_K = 8192
_N = 16384
_BM = 2048
_BN = 256
_NCHUNK = _K // _BN


def _vq_body(iota_ref, x_ref, emb_ref, idx_ref, msum_ref, emb2_ref, e2_ref):
    i = pl.program_id(0)

    @pl.when(i == 0)
    def _():
        emb2_ref[...] = emb_ref[...] + emb_ref[...]
        sq = emb_ref[...] * emb_ref[...]
        e2_ref[...] = lax.dot_general(
            jnp.ones((1, _DIM), jnp.float32), sq, (((1,), (1,)), ((), ())),
            preferred_element_type=jnp.float32)

    x_blk = x_ref[...]
    x2 = jnp.sum(x_blk * x_blk, axis=1, keepdims=True)
    iof = iota_ref[...]

    def scan(add_e2):
        accv = None
        acci = None
        for j in range(_NCHUNK):
            e_blk = emb2_ref[pl.ds(j * _BN, _BN), :]
            mm2 = lax.dot_general(x_blk, e_blk, (((1,), (1,)), ((), ())),
                                  preferred_element_type=jnp.float32)
            if add_e2:
                t = x2 + e2_ref[:, pl.ds(j * _BN, _BN)]
            else:
                t = x2
            s = t - mm2
            s_lo = s[:, : _BN // 2]
            s_hi = s[:, _BN // 2:]
            fold = s_hi < s_lo
            sv = jnp.where(fold, s_hi, s_lo)
            si = jnp.where(fold, iof + jnp.float32(j * _BN + _BN // 2),
                           iof + jnp.float32(j * _BN))
            if accv is None:
                accv = sv
                acci = si
            else:
                take = sv < accv
                accv = jnp.where(take, sv, accv)
                acci = jnp.where(take, si, acci)
        m = jnp.min(accv, axis=1, keepdims=True)
        cand = jnp.where(accv == m, acci, jnp.float32(1e9))
        ci = jnp.min(cand, axis=1, keepdims=True)
        idx_ref[...] = ci[:, 0].astype(jnp.int32)
        part = jnp.sum(m)

        @pl.when(i == 0)
        def _():
            msum_ref[0, 0] = part

        @pl.when(i != 0)
        def _():
            msum_ref[0, 0] = msum_ref[0, 0] + part

    safe = jnp.max(e2_ref[...]) * jnp.float32(1 << 25) < jnp.min(x2)

    @pl.when(safe)
    def _():
        scan(False)

    @pl.when(jnp.logical_not(safe))
    def _():
        scan(True)


def _vq_argmin(iota_f, x, embedding):
    return pl.pallas_call(
        _vq_body,
        grid=(_N // _BM,),
        in_specs=[
            pl.BlockSpec((1, _BN // 2), lambda i: (0, 0)),
            pl.BlockSpec((_BM, _DIM), lambda i: (i, 0)),
            pl.BlockSpec((_K, _DIM), lambda i: (0, 0)),
        ],
        out_specs=[
            pl.BlockSpec((_BM,), lambda i: (i,)),
            pl.BlockSpec((1, 1), lambda i: (0, 0), memory_space=pltpu.SMEM),
        ],
        out_shape=[
            jax.ShapeDtypeStruct((_N,), jnp.int32),
            jax.ShapeDtypeStruct((1, 1), jnp.float32),
        ],
        scratch_shapes=[
            pltpu.VMEM((_K, _DIM), jnp.float32),
            pltpu.VMEM((1, _K), jnp.float32),
        ],
    )(iota_f, x, embedding)


def _sc_gather(embedding, indices):
    info = plsc.get_sparse_core_info()
    nc, ns = info.num_cores, info.num_subcores
    nw = nc * ns
    bpw = _N // nw
    ch = 128
    nch = bpw // ch
    mesh = plsc.VectorSubcoreMesh(core_axis_name="c", subcore_axis_name="s")

    @functools.partial(
        pl.kernel,
        mesh=mesh,
        out_type=jax.ShapeDtypeStruct((_N, _DIM), jnp.float32),
        scratch_types=[
            pltpu.VMEM((bpw,), jnp.int32),
            pltpu.VMEM((ch, _DIM), jnp.float32),
            pltpu.VMEM((ch, _DIM), jnp.float32),
            pltpu.SemaphoreType.DMA,
            pltpu.SemaphoreType.DMA,
        ],
    )
    def gk(table_hbm, idx_hbm, out_hbm, idx_v, rows0, rows1, sem0, sem1):
        wid = lax.axis_index("s") * nc + lax.axis_index("c")
        base = wid * bpw
        pltpu.sync_copy(idx_hbm.at[pl.ds(base, bpw)], idx_v)
        bufs = (rows0, rows1)
        sems = (sem0, sem1)
        prev = pltpu.async_copy(table_hbm.at[idx_v.at[pl.ds(0, ch)]],
                                bufs[0], sems[0])
        for c in range(1, nch):
            nxt = pltpu.async_copy(table_hbm.at[idx_v.at[pl.ds(c * ch, ch)]],
                                   bufs[c % 2], sems[c % 2])
            prev.wait()
            pltpu.sync_copy(bufs[(c - 1) % 2],
                            out_hbm.at[pl.ds(base + (c - 1) * ch, ch)])
            prev = nxt
        prev.wait()
        pltpu.sync_copy(bufs[(nch - 1) % 2],
                        out_hbm.at[pl.ds(base + (nch - 1) * ch, ch)])

    return gk(embedding, indices)


def kernel(x, embedding):
    input_shape = x.shape
    flat_x = x.reshape(-1, _DIM)
    iota_f = jnp.arange(_BN // 2, dtype=jnp.float32)[None, :]
    indices, msum = _vq_argmin(iota_f, flat_x, embedding)
    q = _sc_gather(embedding, indices)
    commitment_loss = msum[0, 0] / (_N * _DIM)
    quantized = q.reshape(input_shape)
    return quantized, indices, commitment_loss
